# baseline (speedup 1.0000x reference)
import functools

import jax
import jax.numpy as jnp
from jax import lax
from jax.experimental import pallas as pl
from jax.experimental.pallas import tpu as pltpu
from jax.experimental.pallas import tpu_sc as plsc

N = 10000
E = 320000
D = 128
DO = 10
WL2 = 16

NC = 2
NS = 16
K = 128
CA = 160
EPAD = NS * CA * K
RPT = 640
NPAD = NS * RPT

BR = 1024
NBUF = 4
WH = 64


def _memset(buf, value, nrows, ncols):
    v = jnp.full((16,), value, jnp.float32)

    def body(i, carry):
        for j in range(ncols // 16):
            buf[i, pl.ds(j * 16, 16)] = v
        return carry

    lax.fori_loop(0, nrows, body, 0)


def _zero_acc_slice(zbuf, acc, sid):
    for j in range(RPT // K):
        pltpu.sync_copy(zbuf, acc.at[pl.ds(sid * RPT + j * K, K)])


def _edge_loop(h_hbm, srcv, dstv, rows, acc, gsem, ssem):
    def body(g, carry):
        base = g * NBUF
        gs = [pltpu.async_copy(h_hbm.at[srcv.at[base + j]], rows.at[j],
                               gsem.at[j]) for j in range(NBUF)]
        ss = []
        for j in range(NBUF):
            gs[j].wait()
            ss.append(pltpu.async_copy(rows.at[j], acc.at[dstv.at[base + j]],
                                       ssem.at[j], add=True))
        for s in ss:
            s.wait()
        return carry

    lax.fori_loop(0, CA // NBUF, body, 0)


def _make_agg_split():
    mesh = plsc.VectorSubcoreMesh(core_axis_name="c", subcore_axis_name="s")

    @functools.partial(
        pl.kernel, mesh=mesh,
        out_type=[jax.ShapeDtypeStruct((NPAD, WH), jnp.float32)] * 2,
        compiler_params=pltpu.CompilerParams(use_tc_tiling_on_sc=False),
        scratch_types=[
            pltpu.VMEM((CA, K), jnp.int32),
            pltpu.VMEM((CA, K), jnp.int32),
            pltpu.VMEM((NBUF, K, WH), jnp.float32),
            pltpu.VMEM_SHARED((NPAD, WH), jnp.float32),
            pltpu.SemaphoreType.DMA((NBUF,)),
            pltpu.SemaphoreType.DMA((NBUF,)),
        ],
    )
    def agg(ha_hbm, hb_hbm, s0_hbm, d0_hbm, outa_hbm, outb_hbm,
            srcv, dstv, rows, acc, gsem, ssem):
        cid = lax.axis_index("c")
        sid = lax.axis_index("s")
        sl = pl.ds(sid * RPT, RPT)

        @pl.when(cid == 0)
        def _():
            pltpu.sync_copy(s0_hbm.at[sid], srcv)
            pltpu.sync_copy(d0_hbm.at[sid], dstv)
            for h_hbm, out_hbm in ((ha_hbm, outa_hbm), (hb_hbm, outb_hbm)):
                _memset(rows.at[0], 0.0, K, WH)
                _zero_acc_slice(rows.at[0], acc, sid)
                plsc.subcore_barrier()
                _edge_loop(h_hbm, srcv, dstv, rows, acc, gsem, ssem)
                plsc.subcore_barrier()
                pltpu.sync_copy(acc.at[sl], out_hbm.at[sl])
                plsc.subcore_barrier()

    return agg


def _make_agg(W):
    mesh = plsc.VectorSubcoreMesh(core_axis_name="c", subcore_axis_name="s")

    @functools.partial(
        pl.kernel, mesh=mesh,
        out_type=jax.ShapeDtypeStruct((NPAD, W), jnp.float32),
        compiler_params=pltpu.CompilerParams(use_tc_tiling_on_sc=False),
        scratch_types=[
            pltpu.VMEM((CA, K), jnp.int32),
            pltpu.VMEM((CA, K), jnp.int32),
            pltpu.VMEM((NBUF, K, W), jnp.float32),
            pltpu.VMEM_SHARED((NPAD, W), jnp.float32),
            pltpu.SemaphoreType.DMA((NBUF,)),
            pltpu.SemaphoreType.DMA((NBUF,)),
        ],
    )
    def agg(h_hbm, s0_hbm, d0_hbm, out_hbm, srcv, dstv, rows, acc,
            gsem, ssem):
        cid = lax.axis_index("c")
        sid = lax.axis_index("s")
        sl = pl.ds(sid * RPT, RPT)

        @pl.when(cid == 0)
        def _():
            pltpu.sync_copy(s0_hbm.at[sid], srcv)
            pltpu.sync_copy(d0_hbm.at[sid], dstv)
            _memset(rows.at[0], 0.0, K, W)
            _zero_acc_slice(rows.at[0], acc, sid)
            plsc.subcore_barrier()
            _edge_loop(h_hbm, srcv, dstv, rows, acc, gsem, ssem)
            plsc.subcore_barrier()
            pltpu.sync_copy(acc.at[sl], out_hbm.at[sl])

    return agg


def _make_deg():
    mesh = plsc.VectorSubcoreMesh(core_axis_name="c", subcore_axis_name="s")

    @functools.partial(
        pl.kernel, mesh=mesh,
        out_type=jax.ShapeDtypeStruct((NPAD, WL2), jnp.float32),
        compiler_params=pltpu.CompilerParams(use_tc_tiling_on_sc=False),
        scratch_types=[
            pltpu.VMEM((CA, K), jnp.int32),
            pltpu.VMEM((K, WL2), jnp.float32),
            pltpu.VMEM((K, WL2), jnp.float32),
            pltpu.VMEM_SHARED((NPAD, WL2), jnp.float32),
            pltpu.SemaphoreType.DMA((NBUF,)),
        ],
    )
    def deg(d0_hbm, out_hbm, dstv, ones_v, zbuf, acc, ssem):
        cid = lax.axis_index("c")
        sid = lax.axis_index("s")
        sl = pl.ds(sid * RPT, RPT)

        @pl.when(cid == 0)
        def _():
            pltpu.sync_copy(d0_hbm.at[sid], dstv)
            _memset(ones_v, 1.0, K, WL2)
            _memset(zbuf, 0.0, K, WL2)
            _zero_acc_slice(zbuf, acc, sid)
            plsc.subcore_barrier()

            def body(g, carry):
                base = g * NBUF
                ss = [pltpu.async_copy(ones_v, acc.at[dstv.at[base + j]],
                                       ssem.at[j], add=True)
                      for j in range(NBUF)]
                for s in ss:
                    s.wait()
                return carry

            lax.fori_loop(0, CA // NBUF, body, 0)
            plsc.subcore_barrier()
            pltpu.sync_copy(acc.at[sl], out_hbm.at[sl])

    return deg


_agg1 = _make_agg_split()
_agg16 = _make_agg(WL2)
_deg = _make_deg()


def _tc_a_body(degp_ref, x_ref, w1_ref, ha_ref, hb_ref, dis_ref):
    deg = degp_ref[:, 0:1] + 1.0
    dis = lax.rsqrt(deg)
    h = jnp.dot(x_ref[...], w1_ref[...], preferred_element_type=jnp.float32)
    hs = h * dis
    ha_ref[...] = hs[:, :WH]
    hb_ref[...] = hs[:, WH:]
    dis_ref[...] = jnp.broadcast_to(dis, dis_ref.shape)


def _tc_a(degp, xpad, W1):
    return pl.pallas_call(
        _tc_a_body,
        grid=(NPAD // BR,),
        in_specs=[
            pl.BlockSpec((BR, WL2), lambda i: (i, 0)),
            pl.BlockSpec((BR, D), lambda i: (i, 0)),
            pl.BlockSpec((D, D), lambda i: (0, 0)),
        ],
        out_specs=[
            pl.BlockSpec((BR, WH), lambda i: (i, 0)),
            pl.BlockSpec((BR, WH), lambda i: (i, 0)),
            pl.BlockSpec((BR, 8), lambda i: (i, 0)),
        ],
        out_shape=[
            jax.ShapeDtypeStruct((NPAD, WH), jnp.float32),
            jax.ShapeDtypeStruct((NPAD, WH), jnp.float32),
            jax.ShapeDtypeStruct((NPAD, 8), jnp.float32),
        ],
    )(degp, xpad, W1)


def _tc_b_body(pa_ref, pb_ref, ha_ref, hb_ref, dis_ref, b1_ref, w2_ref,
               h2_ref):
    dis = dis_ref[:, 0:1]
    sa = pa_ref[...] + ha_ref[...]
    sb = pb_ref[...] + hb_ref[...]
    s = jnp.concatenate([sa, sb], axis=1)
    z = jnp.maximum(s * dis + b1_ref[...], 0.0)
    h2 = jnp.dot(z, w2_ref[...], preferred_element_type=jnp.float32)
    h2_ref[...] = h2 * dis


def _tc_b(parta, partb, h1a, h1b, dis, b1row, W2p):
    return pl.pallas_call(
        _tc_b_body,
        grid=(NPAD // BR,),
        in_specs=[
            pl.BlockSpec((BR, WH), lambda i: (i, 0)),
            pl.BlockSpec((BR, WH), lambda i: (i, 0)),
            pl.BlockSpec((BR, WH), lambda i: (i, 0)),
            pl.BlockSpec((BR, WH), lambda i: (i, 0)),
            pl.BlockSpec((BR, 8), lambda i: (i, 0)),
            pl.BlockSpec((1, D), lambda i: (0, 0)),
            pl.BlockSpec((D, WL2), lambda i: (0, 0)),
        ],
        out_specs=pl.BlockSpec((BR, WL2), lambda i: (i, 0)),
        out_shape=jax.ShapeDtypeStruct((NPAD, WL2), jnp.float32),
    )(parta, partb, h1a, h1b, dis, b1row, W2p)


def _tc_c_body(part_ref, h2_ref, dis_ref, b2_ref, out_ref):
    dis = dis_ref[:, 0:1]
    s = part_ref[...] + h2_ref[...]
    out_ref[...] = s * dis + b2_ref[...]


def _tc_c(part2, h2p, dis, b2row):
    return pl.pallas_call(
        _tc_c_body,
        grid=(NPAD // BR,),
        in_specs=[
            pl.BlockSpec((BR, WL2), lambda i: (i, 0)),
            pl.BlockSpec((BR, WL2), lambda i: (i, 0)),
            pl.BlockSpec((BR, 8), lambda i: (i, 0)),
            pl.BlockSpec((1, WL2), lambda i: (0, 0)),
        ],
        out_specs=pl.BlockSpec((BR, WL2), lambda i: (i, 0)),
        out_shape=jax.ShapeDtypeStruct((NPAD, WL2), jnp.float32),
    )(part2, h2p, dis, b2row)


def kernel(x, edge_index, W1, b1, W2, b2):
    src = edge_index[0]
    dst = edge_index[1]
    pad_idx = jnp.full((EPAD - E,), N, jnp.int32)
    s0 = jnp.concatenate([src, pad_idx]).reshape(NS, CA, K)
    d0 = jnp.concatenate([dst, pad_idx]).reshape(NS, CA, K)
    xpad = jnp.pad(x, ((0, NPAD - N), (0, 0)))

    W2p = jnp.pad(W2, ((0, 0), (0, WL2 - DO)))
    b1row = b1[None, :]
    b2row = jnp.pad(b2, (0, WL2 - DO))[None, :]

    degp = _deg(d0)
    h1a, h1b, dis = _tc_a(degp, xpad, W1)
    parta, partb = _agg1(h1a, h1b, s0, d0)
    h2p = _tc_b(parta, partb, h1a, h1b, dis, b1row, W2p)
    part2 = _agg16(h2p, s0, d0)
    outp = _tc_c(part2, h2p, dis, b2row)
    return outp[:N, :DO]


# speedup vs baseline: 1.4576x; 1.4576x over previous
import functools

import jax
import jax.numpy as jnp
from jax import lax
from jax.experimental import pallas as pl
from jax.experimental.pallas import tpu as pltpu
from jax.experimental.pallas import tpu_sc as plsc

N = 10000
E = 320000
D = 128
DO = 10
WL2 = 16

NC = 2
NS = 16
NTILES = NC * NS
K = 128
CA = 144
CB = 16
EPAD = NS * (CA + CB) * K
RPT = 640
NPAD = NS * RPT

BR = 1024
NBUF = 4
WH = 64


def _edge_loop(h_hbm, srcv, dstv, rows, acc, gsem, ssem, ngroups):
    def body(g, carry):
        base = g * NBUF
        gs = [pltpu.async_copy(h_hbm.at[srcv.at[base + j]], rows.at[j],
                               gsem.at[j]) for j in range(NBUF)]
        ss = []
        for j in range(NBUF):
            gs[j].wait()
            ss.append(pltpu.async_copy(rows.at[j], acc.at[dstv.at[base + j]],
                                       ssem.at[j], add=True))
        for s in ss:
            s.wait()
        return carry

    lax.fori_loop(0, ngroups, body, 0)


def _memset(buf, value, nrows, ncols):
    v = jnp.full((16,), value, jnp.float32)

    def body(i, carry):
        for j in range(ncols // 16):
            buf[i, pl.ds(j * 16, 16)] = v
        return carry

    lax.fori_loop(0, nrows, body, 0)


def _zero_acc_slice(zbuf, acc, sid):
    for j in range(RPT // K):
        pltpu.sync_copy(zbuf, acc.at[pl.ds(sid * RPT + j * K, K)])


def _load_idx(cid, sid, s0_hbm, d0_hbm, s1_hbm, d1_hbm, srcv, dstv):
    @pl.when(cid == 0)
    def _():
        pltpu.sync_copy(s0_hbm.at[sid], srcv)
        pltpu.sync_copy(d0_hbm.at[sid], dstv)

    @pl.when(cid == 1)
    def _():
        pltpu.sync_copy(s1_hbm.at[sid], srcv.at[pl.ds(0, CB)])
        pltpu.sync_copy(d1_hbm.at[sid], dstv.at[pl.ds(0, CB)])


def _make_agg_split():
    mesh = plsc.VectorSubcoreMesh(core_axis_name="c", subcore_axis_name="s")

    @functools.partial(
        pl.kernel, mesh=mesh,
        out_type=[jax.ShapeDtypeStruct((NC, NPAD, WH), jnp.float32)] * 2,
        compiler_params=pltpu.CompilerParams(use_tc_tiling_on_sc=False),
        scratch_types=[
            pltpu.VMEM((CA, K), jnp.int32),
            pltpu.VMEM((CA, K), jnp.int32),
            pltpu.VMEM((NBUF, K, WH), jnp.float32),
            pltpu.VMEM_SHARED((NPAD, WH), jnp.float32),
            pltpu.SemaphoreType.DMA((NBUF,)),
            pltpu.SemaphoreType.DMA((NBUF,)),
        ],
    )
    def agg(ha_hbm, hb_hbm, s0_hbm, d0_hbm, s1_hbm, d1_hbm,
            outa_hbm, outb_hbm, srcv, dstv, rows, acc, gsem, ssem):
        cid = lax.axis_index("c")
        sid = lax.axis_index("s")
        sl = pl.ds(sid * RPT, RPT)
        _load_idx(cid, sid, s0_hbm, d0_hbm, s1_hbm, d1_hbm, srcv, dstv)
        ngroups = jnp.where(cid == 0, CA // NBUF, CB // NBUF)
        for h_hbm, out_hbm in ((ha_hbm, outa_hbm), (hb_hbm, outb_hbm)):
            _memset(rows.at[0], 0.0, K, WH)
            _zero_acc_slice(rows.at[0], acc, sid)
            plsc.subcore_barrier()
            _edge_loop(h_hbm, srcv, dstv, rows, acc, gsem, ssem, ngroups)
            plsc.subcore_barrier()
            pltpu.sync_copy(acc.at[sl], out_hbm.at[cid, sl])
            plsc.subcore_barrier()

    return agg


def _make_agg(W):
    mesh = plsc.VectorSubcoreMesh(core_axis_name="c", subcore_axis_name="s")

    @functools.partial(
        pl.kernel, mesh=mesh,
        out_type=jax.ShapeDtypeStruct((NC, NPAD, W), jnp.float32),
        compiler_params=pltpu.CompilerParams(use_tc_tiling_on_sc=False),
        scratch_types=[
            pltpu.VMEM((CA, K), jnp.int32),
            pltpu.VMEM((CA, K), jnp.int32),
            pltpu.VMEM((NBUF, K, W), jnp.float32),
            pltpu.VMEM_SHARED((NPAD, W), jnp.float32),
            pltpu.SemaphoreType.DMA((NBUF,)),
            pltpu.SemaphoreType.DMA((NBUF,)),
        ],
    )
    def agg(h_hbm, s0_hbm, d0_hbm, s1_hbm, d1_hbm, out_hbm,
            srcv, dstv, rows, acc, gsem, ssem):
        cid = lax.axis_index("c")
        sid = lax.axis_index("s")
        sl = pl.ds(sid * RPT, RPT)
        _load_idx(cid, sid, s0_hbm, d0_hbm, s1_hbm, d1_hbm, srcv, dstv)
        ngroups = jnp.where(cid == 0, CA // NBUF, CB // NBUF)
        _memset(rows.at[0], 0.0, K, W)
        _zero_acc_slice(rows.at[0], acc, sid)
        plsc.subcore_barrier()
        _edge_loop(h_hbm, srcv, dstv, rows, acc, gsem, ssem, ngroups)
        plsc.subcore_barrier()
        pltpu.sync_copy(acc.at[sl], out_hbm.at[cid, sl])

    return agg


def _make_deg():
    mesh = plsc.VectorSubcoreMesh(core_axis_name="c", subcore_axis_name="s")

    @functools.partial(
        pl.kernel, mesh=mesh,
        out_type=jax.ShapeDtypeStruct((NC, NPAD, WL2), jnp.float32),
        compiler_params=pltpu.CompilerParams(use_tc_tiling_on_sc=False),
        scratch_types=[
            pltpu.VMEM((CA, K), jnp.int32),
            pltpu.VMEM((K, WL2), jnp.float32),
            pltpu.VMEM((K, WL2), jnp.float32),
            pltpu.VMEM_SHARED((NPAD, WL2), jnp.float32),
            pltpu.SemaphoreType.DMA((NBUF,)),
        ],
    )
    def deg(d0_hbm, d1_hbm, out_hbm, dstv, ones_v, zbuf, acc, ssem):
        cid = lax.axis_index("c")
        sid = lax.axis_index("s")
        sl = pl.ds(sid * RPT, RPT)

        @pl.when(cid == 0)
        def _():
            pltpu.sync_copy(d0_hbm.at[sid], dstv)

        @pl.when(cid == 1)
        def _():
            pltpu.sync_copy(d1_hbm.at[sid], dstv.at[pl.ds(0, CB)])

        ngroups = jnp.where(cid == 0, CA // NBUF, CB // NBUF)
        _memset(ones_v, 1.0, K, WL2)
        _memset(zbuf, 0.0, K, WL2)
        _zero_acc_slice(zbuf, acc, sid)
        plsc.subcore_barrier()

        def body(g, carry):
            base = g * NBUF
            ss = [pltpu.async_copy(ones_v, acc.at[dstv.at[base + j]],
                                   ssem.at[j], add=True) for j in range(NBUF)]
            for s in ss:
                s.wait()
            return carry

        lax.fori_loop(0, ngroups, body, 0)
        plsc.subcore_barrier()
        pltpu.sync_copy(acc.at[sl], out_hbm.at[cid, sl])

    return deg


_agg1 = _make_agg_split()
_agg16 = _make_agg(WL2)
_deg = _make_deg()


def _tc_a_body(degp_ref, x_ref, w1_ref, ha_ref, hb_ref, dis_ref):
    deg = degp_ref[0, :, 0:1] + degp_ref[1, :, 0:1] + 1.0
    dis = lax.rsqrt(deg)
    h = jnp.dot(x_ref[...], w1_ref[...], preferred_element_type=jnp.float32)
    hs = h * dis
    ha_ref[...] = hs[:, :WH]
    hb_ref[...] = hs[:, WH:]
    dis_ref[...] = jnp.broadcast_to(dis, dis_ref.shape)


def _tc_a(degp, xpad, W1):
    return pl.pallas_call(
        _tc_a_body,
        grid=(NPAD // BR,),
        in_specs=[
            pl.BlockSpec((2, BR, WL2), lambda i: (0, i, 0)),
            pl.BlockSpec((BR, D), lambda i: (i, 0)),
            pl.BlockSpec((D, D), lambda i: (0, 0)),
        ],
        out_specs=[
            pl.BlockSpec((BR, WH), lambda i: (i, 0)),
            pl.BlockSpec((BR, WH), lambda i: (i, 0)),
            pl.BlockSpec((BR, 8), lambda i: (i, 0)),
        ],
        out_shape=[
            jax.ShapeDtypeStruct((NPAD, WH), jnp.float32),
            jax.ShapeDtypeStruct((NPAD, WH), jnp.float32),
            jax.ShapeDtypeStruct((NPAD, 8), jnp.float32),
        ],
    )(degp, xpad, W1)


def _tc_b_body(pa_ref, pb_ref, ha_ref, hb_ref, dis_ref, b1_ref, w2_ref,
               h2_ref):
    dis = dis_ref[:, 0:1]
    sa = pa_ref[0] + pa_ref[1] + ha_ref[...]
    sb = pb_ref[0] + pb_ref[1] + hb_ref[...]
    s = jnp.concatenate([sa, sb], axis=1)
    z = jnp.maximum(s * dis + b1_ref[...], 0.0)
    h2 = jnp.dot(z, w2_ref[...], preferred_element_type=jnp.float32)
    h2_ref[...] = h2 * dis


def _tc_b(parta, partb, h1a, h1b, dis, b1row, W2p):
    return pl.pallas_call(
        _tc_b_body,
        grid=(NPAD // BR,),
        in_specs=[
            pl.BlockSpec((2, BR, WH), lambda i: (0, i, 0)),
            pl.BlockSpec((2, BR, WH), lambda i: (0, i, 0)),
            pl.BlockSpec((BR, WH), lambda i: (i, 0)),
            pl.BlockSpec((BR, WH), lambda i: (i, 0)),
            pl.BlockSpec((BR, 8), lambda i: (i, 0)),
            pl.BlockSpec((1, D), lambda i: (0, 0)),
            pl.BlockSpec((D, WL2), lambda i: (0, 0)),
        ],
        out_specs=pl.BlockSpec((BR, WL2), lambda i: (i, 0)),
        out_shape=jax.ShapeDtypeStruct((NPAD, WL2), jnp.float32),
    )(parta, partb, h1a, h1b, dis, b1row, W2p)


def _tc_c_body(part_ref, h2_ref, dis_ref, b2_ref, out_ref):
    dis = dis_ref[:, 0:1]
    s = part_ref[0] + part_ref[1] + h2_ref[...]
    out_ref[...] = s * dis + b2_ref[...]


def _tc_c(part2, h2p, dis, b2row):
    return pl.pallas_call(
        _tc_c_body,
        grid=(NPAD // BR,),
        in_specs=[
            pl.BlockSpec((2, BR, WL2), lambda i: (0, i, 0)),
            pl.BlockSpec((BR, WL2), lambda i: (i, 0)),
            pl.BlockSpec((BR, 8), lambda i: (i, 0)),
            pl.BlockSpec((1, WL2), lambda i: (0, 0)),
        ],
        out_specs=pl.BlockSpec((BR, WL2), lambda i: (i, 0)),
        out_shape=jax.ShapeDtypeStruct((NPAD, WL2), jnp.float32),
    )(part2, h2p, dis, b2row)


def kernel(x, edge_index, W1, b1, W2, b2):
    src = edge_index[0]
    dst = edge_index[1]
    pad_idx = jnp.full((EPAD - E,), N, jnp.int32)
    e0 = NS * CA * K
    srcpad = jnp.concatenate([src, pad_idx])
    dstpad = jnp.concatenate([dst, pad_idx])
    s0 = srcpad[:e0].reshape(NS, CA, K)
    s1 = srcpad[e0:].reshape(NS, CB, K)
    d0 = dstpad[:e0].reshape(NS, CA, K)
    d1 = dstpad[e0:].reshape(NS, CB, K)
    xpad = jnp.pad(x, ((0, NPAD - N), (0, 0)))

    W2p = jnp.pad(W2, ((0, 0), (0, WL2 - DO)))
    b1row = b1[None, :]
    b2row = jnp.pad(b2, (0, WL2 - DO))[None, :]

    degp = _deg(d0, d1)
    h1a, h1b, dis = _tc_a(degp, xpad, W1)
    parta, partb = _agg1(h1a, h1b, s0, d0, s1, d1)
    h2p = _tc_b(parta, partb, h1a, h1b, dis, b1row, W2p)
    part2 = _agg16(h2p, s0, d0, s1, d1)
    outp = _tc_c(part2, h2p, dis, b2row)
    return outp[:N, :DO]
